# baseline (speedup 1.0000x reference)
import functools

import jax
import jax.numpy as jnp
from jax import lax
from jax.experimental import pallas as pl
from jax.experimental.pallas import tpu as pltpu
from jax.experimental.pallas import tpu_sc as plsc

B = 16
L = 512
D = 256
NH = 4
DH = 64
T = 1000
TP = 1024
TPW = TP // 2
NW = 32
NCH = 4
CH = (B * L) // (NW * NCH)


@functools.partial(
    pl.kernel,
    mesh=plsc.VectorSubcoreMesh(core_axis_name="c", subcore_axis_name="s"),
    out_type=jax.ShapeDtypeStruct((B * L, TPW), jnp.int32),
    scratch_types=[
        pltpu.VMEM((NCH, CH), jnp.int32),
        pltpu.VMEM((2, CH, TPW), jnp.int32),
        pltpu.SemaphoreType.DMA,
        pltpu.SemaphoreType.DMA,
    ],
)
def _sc_row_gather(A_hbm, idx_hbm, out_hbm, idx_v, rows_v, sem0, sem1):
    wid = lax.axis_index("s") * 2 + lax.axis_index("c")
    pltpu.sync_copy(idx_hbm.at[wid], idx_v)
    sems = (sem0, sem1)
    copies = [None, None]
    copies[0] = pltpu.async_copy(A_hbm.at[idx_v.at[0]], rows_v.at[0], sems[0])
    for j in range(NCH):
        p = j % 2
        if j + 1 < NCH:
            pn = (j + 1) % 2
            copies[pn] = pltpu.async_copy(
                A_hbm.at[idx_v.at[j + 1]], rows_v.at[pn], sems[pn])
        copies[p].wait()
        base = wid * (NCH * CH) + j * CH
        pltpu.sync_copy(rows_v.at[p], out_hbm.at[pl.ds(base, CH)])


def _tc_body(t_ref, ar_ref, x_ref, u_ref, Wb0_ref, Wu0_ref, b0_ref,
             Wb1_ref, Wu1_ref, b1_ref, out_ref):
    t_idx = t_ref[0, 0, :]
    Ar = ar_ref[0]
    x = x_ref[0]
    u = u_ref[0]

    rows = lax.broadcasted_iota(jnp.int32, (TP, L), 0)
    Pt = (rows == t_idx[None, :]).astype(jnp.bfloat16)

    Asub = jnp.dot(Ar, Pt, preferred_element_type=jnp.float32)
    Asub_b = Asub.astype(jnp.bfloat16)

    r2 = lax.broadcasted_iota(jnp.int32, (L, L), 0)
    c2 = lax.broadcasted_iota(jnp.int32, (L, L), 1)
    diag = jnp.sum(jnp.where(r2 == c2, Asub, 0.0), axis=0)
    S = jnp.sum(diag)
    deg = jnp.sum(Asub, axis=1) + S + 1e-8
    inv_deg = (1.0 / deg)[:, None]

    ub = u.astype(jnp.bfloat16)

    def layer(xin, Wb_ref, Wu_ref, b_ref):
        cd = jnp.sum(diag[:, None] * xin, axis=0)
        g = jnp.dot(Asub_b, xin.astype(jnp.bfloat16),
                    preferred_element_type=jnp.float32)
        msg = (g + cd[None, :]) * inv_deg
        h = jnp.dot(msg.astype(jnp.bfloat16), Wb_ref[...],
                    preferred_element_type=jnp.float32)
        U = jnp.dot(ub, Wu_ref[...],
                    preferred_element_type=jnp.float32) + b_ref[0][None, :]
        return jnp.maximum(h + U, 0.0) + xin

    x1 = layer(x, Wb0_ref, Wu0_ref, b0_ref)
    x2 = layer(x1, Wb1_ref, Wu1_ref, b1_ref)
    out_ref[0, 0, :] = jnp.mean(x2, axis=0)


def kernel(user_id, event_type, enc_output, user_output, adjacent_matrix,
           W0, Wu0, b0, W1, Wu1, b1):
    t = event_type.astype(jnp.int32) - 1
    t = jnp.where(t < 0, t + 1000, t)
    A_bf = jnp.pad(adjacent_matrix.astype(jnp.bfloat16),
                   ((0, 0), (0, TP - T)))
    A_packed = lax.bitcast_convert_type(
        A_bf.reshape(T, TPW, 2), jnp.int32)

    arows = _sc_row_gather(A_packed, t.reshape(NW, NCH, CH))
    arows = lax.bitcast_convert_type(
        arows, jnp.bfloat16).reshape(B, L, TP)

    t3 = t.reshape(B, 1, L)
    b0r = b0.reshape(1, D)
    b1r = b1.reshape(1, D)
    hmask = (jnp.arange(NH)[:, None, None, None] ==
             jnp.arange(NH)[None, None, :, None])
    Wb0 = jnp.where(hmask, W0[:, :, None, :], 0.0)
    Wb0 = Wb0.reshape(D, D).astype(jnp.bfloat16)
    Wb1 = jnp.where(hmask, W1[:, :, None, :], 0.0)
    Wb1 = Wb1.reshape(D, D).astype(jnp.bfloat16)
    Wu0b = Wu0.astype(jnp.bfloat16)
    Wu1b = Wu1.astype(jnp.bfloat16)

    grid = (B,)
    out = pl.pallas_call(
        _tc_body,
        grid=grid,
        in_specs=[
            pl.BlockSpec((1, 1, L), lambda b: (b, 0, 0)),
            pl.BlockSpec((1, L, TP), lambda b: (b, 0, 0)),
            pl.BlockSpec((1, L, D), lambda b: (b, 0, 0)),
            pl.BlockSpec((1, L, D), lambda b: (b, 0, 0)),
            pl.BlockSpec((D, D), lambda b: (0, 0)),
            pl.BlockSpec((D, D), lambda b: (0, 0)),
            pl.BlockSpec((1, D), lambda b: (0, 0)),
            pl.BlockSpec((D, D), lambda b: (0, 0)),
            pl.BlockSpec((D, D), lambda b: (0, 0)),
            pl.BlockSpec((1, D), lambda b: (0, 0)),
        ],
        out_specs=pl.BlockSpec((1, 1, D), lambda b: (b, 0, 0)),
        out_shape=jax.ShapeDtypeStruct((B, 1, D), jnp.float32),
    )(t3, arows, enc_output, user_output, Wb0, Wu0b, b0r, Wb1, Wu1b, b1r)
    return out.reshape(B, D)


# speedup vs baseline: 2.6889x; 2.6889x over previous
import functools

import jax
import jax.numpy as jnp
from jax import lax
from jax.experimental import pallas as pl
from jax.experimental.pallas import tpu as pltpu
from jax.experimental.pallas import tpu_sc as plsc

B = 16
L = 512
D = 256
NH = 4
DH = 64
T = 1000
TP = 1024
NW = 32
NCH = 8
CH = (B * L) // (NW * NCH)


@functools.partial(
    pl.kernel,
    mesh=plsc.VectorSubcoreMesh(core_axis_name="c", subcore_axis_name="s"),
    out_type=jax.ShapeDtypeStruct((B * L, TP), jnp.float32),
    scratch_types=[
        pltpu.VMEM((NCH, CH), jnp.int32),
        pltpu.VMEM((2, CH, TP), jnp.float32),
        pltpu.SemaphoreType.DMA,
        pltpu.SemaphoreType.DMA,
    ],
)
def _sc_row_gather(A_hbm, idx_hbm, out_hbm, idx_v, rows_v, sem0, sem1):
    wid = lax.axis_index("s") * 2 + lax.axis_index("c")
    pltpu.sync_copy(idx_hbm.at[wid], idx_v)
    sems = (sem0, sem1)
    copies = [None, None]
    copies[0] = pltpu.async_copy(A_hbm.at[idx_v.at[0]], rows_v.at[0], sems[0])
    for j in range(NCH):
        p = j % 2
        if j + 1 < NCH:
            pn = (j + 1) % 2
            copies[pn] = pltpu.async_copy(
                A_hbm.at[idx_v.at[j + 1]], rows_v.at[pn], sems[pn])
        copies[p].wait()
        base = wid * (NCH * CH) + j * CH
        pltpu.sync_copy(rows_v.at[p], out_hbm.at[pl.ds(base, CH)])


def _tc_body(t_ref, ar_ref, x_ref, u_ref, Wb0_ref, Wu0_ref, b0_ref,
             Wb1_ref, Wu1_ref, b1_ref, out_ref):
    t_idx = t_ref[0, 0, :]
    Ar = ar_ref[0].astype(jnp.bfloat16)
    x = x_ref[0]
    u = u_ref[0]

    rows = lax.broadcasted_iota(jnp.int32, (TP, L), 0)
    Pt = (rows == t_idx[None, :]).astype(jnp.bfloat16)

    Asub = jnp.dot(Ar, Pt, preferred_element_type=jnp.float32)
    Asub_b = Asub.astype(jnp.bfloat16)

    r2 = lax.broadcasted_iota(jnp.int32, (L, L), 0)
    c2 = lax.broadcasted_iota(jnp.int32, (L, L), 1)
    diag = jnp.sum(jnp.where(r2 == c2, Asub, 0.0), axis=0)
    S = jnp.sum(diag)
    deg = jnp.sum(Asub, axis=1) + S + 1e-8
    inv_deg = (1.0 / deg)[:, None]

    ub = u.astype(jnp.bfloat16)

    def layer(xin, Wb_ref, Wu_ref, b_ref):
        cd = jnp.sum(diag[:, None] * xin, axis=0)
        g = jnp.dot(Asub_b, xin.astype(jnp.bfloat16),
                    preferred_element_type=jnp.float32)
        msg = (g + cd[None, :]) * inv_deg
        h = jnp.dot(msg.astype(jnp.bfloat16), Wb_ref[...],
                    preferred_element_type=jnp.float32)
        U = jnp.dot(ub, Wu_ref[...],
                    preferred_element_type=jnp.float32) + b_ref[0][None, :]
        return jnp.maximum(h + U, 0.0) + xin

    x1 = layer(x, Wb0_ref, Wu0_ref, b0_ref)
    x2 = layer(x1, Wb1_ref, Wu1_ref, b1_ref)
    out_ref[0, 0, :] = jnp.mean(x2, axis=0)


def kernel(user_id, event_type, enc_output, user_output, adjacent_matrix,
           W0, Wu0, b0, W1, Wu1, b1):
    t = event_type.astype(jnp.int32) - 1
    t = jnp.where(t < 0, t + 1000, t)
    A_pad = jnp.pad(adjacent_matrix, ((0, 0), (0, TP - T)))

    arows = _sc_row_gather(A_pad, t.reshape(NW, NCH, CH))
    arows = arows.reshape(B, L, TP)

    t3 = t.reshape(B, 1, L)
    b0r = b0.reshape(1, D)
    b1r = b1.reshape(1, D)
    hmask = (jnp.arange(NH)[:, None, None, None] ==
             jnp.arange(NH)[None, None, :, None])
    Wb0 = jnp.where(hmask, W0[:, :, None, :], 0.0)
    Wb0 = Wb0.reshape(D, D).astype(jnp.bfloat16)
    Wb1 = jnp.where(hmask, W1[:, :, None, :], 0.0)
    Wb1 = Wb1.reshape(D, D).astype(jnp.bfloat16)
    Wu0b = Wu0.astype(jnp.bfloat16)
    Wu1b = Wu1.astype(jnp.bfloat16)

    grid = (B,)
    out = pl.pallas_call(
        _tc_body,
        grid=grid,
        in_specs=[
            pl.BlockSpec((1, 1, L), lambda b: (b, 0, 0)),
            pl.BlockSpec((1, L, TP), lambda b: (b, 0, 0)),
            pl.BlockSpec((1, L, D), lambda b: (b, 0, 0)),
            pl.BlockSpec((1, L, D), lambda b: (b, 0, 0)),
            pl.BlockSpec((D, D), lambda b: (0, 0)),
            pl.BlockSpec((D, D), lambda b: (0, 0)),
            pl.BlockSpec((1, D), lambda b: (0, 0)),
            pl.BlockSpec((D, D), lambda b: (0, 0)),
            pl.BlockSpec((D, D), lambda b: (0, 0)),
            pl.BlockSpec((1, D), lambda b: (0, 0)),
        ],
        out_specs=pl.BlockSpec((1, 1, D), lambda b: (b, 0, 0)),
        out_shape=jax.ShapeDtypeStruct((B, 1, D), jnp.float32),
    )(t3, arows, enc_output, user_output, Wb0, Wu0b, b0r, Wb1, Wu1b, b1r)
    return out.reshape(B, D)
